# Initial kernel scaffold; baseline (speedup 1.0000x reference)
import functools

import jax
import jax.numpy as jnp
from jax import lax
from jax.experimental import pallas as pl
from jax.experimental.pallas import tpu as pltpu
from jax.experimental.pallas import tpu_sc as plsc

B = 16384
L = 50
EMBD = 64
NTOT = B * L
NW = 32
B_PER_W = NTOT // NW
CH = 800
NCHUNK = B_PER_W // CH

_mesh = plsc.VectorSubcoreMesh(core_axis_name="c", subcore_axis_name="s")


@functools.partial(
    pl.kernel,
    mesh=_mesh,
    out_type=jax.ShapeDtypeStruct((NTOT, EMBD), jnp.float32),
    scratch_types=[
        pltpu.VMEM((CH,), jnp.int32),
        pltpu.VMEM((CH, EMBD), jnp.float32),
        pltpu.SemaphoreType.DMA,
    ],
)
def _gather_kernel(idx_hbm, table_hbm, out_hbm, idx_v, rows_v, sem):
    wid = lax.axis_index("s") * 2 + lax.axis_index("c")
    base = wid * B_PER_W

    def step(i, carry):
        off = base + i * CH
        pltpu.sync_copy(idx_hbm.at[pl.ds(off, CH)], idx_v)
        pltpu.async_copy(table_hbm.at[idx_v], rows_v, sem).wait()
        pltpu.sync_copy(rows_v, out_hbm.at[pl.ds(off, CH)])
        return carry

    lax.fori_loop(0, NCHUNK, step, 0)


def kernel(x, table):
    idx = x.reshape(NTOT)
    out = _gather_kernel(idx, table)
    return out.reshape(B, L, EMBD)


# speedup vs baseline: 1.8294x; 1.8294x over previous
import functools

import jax
import jax.numpy as jnp
from jax import lax
from jax.experimental import pallas as pl
from jax.experimental.pallas import tpu as pltpu
from jax.experimental.pallas import tpu_sc as plsc

B = 16384
L = 50
EMBD = 64
NTOT = B * L
NW = 32
B_PER_W = NTOT // NW
CH = 800
NCHUNK = B_PER_W // CH

_mesh = plsc.VectorSubcoreMesh(core_axis_name="c", subcore_axis_name="s")


@functools.partial(
    pl.kernel,
    mesh=_mesh,
    out_type=jax.ShapeDtypeStruct((NTOT, EMBD), jnp.float32),
    scratch_types=[
        pltpu.VMEM((CH,), jnp.int32),
        pltpu.VMEM((CH, EMBD), jnp.float32),
        pltpu.SemaphoreType.DMA,
    ],
    compiler_params=pltpu.CompilerParams(use_tc_tiling_on_sc=False),
)
def _gather_kernel(idx_hbm, table_hbm, out_hbm, idx_v, rows_v, sem):
    wid = lax.axis_index("s") * 2 + lax.axis_index("c")
    base = wid * B_PER_W

    def step(i, carry):
        off = base + i * CH
        pltpu.sync_copy(idx_hbm.at[pl.ds(off, CH)], idx_v)
        pltpu.async_copy(table_hbm.at[idx_v], rows_v, sem).wait()
        pltpu.sync_copy(rows_v, out_hbm.at[pl.ds(off, CH)])
        return carry

    lax.fori_loop(0, NCHUNK, step, 0)


def kernel(x, table):
    idx = x.reshape(NTOT)
    out = _gather_kernel(idx, table)
    return out.reshape(B, L, EMBD)


# speedup vs baseline: 1.8900x; 1.0331x over previous
import functools

import jax
import jax.numpy as jnp
from jax import lax
from jax.experimental import pallas as pl
from jax.experimental.pallas import tpu as pltpu
from jax.experimental.pallas import tpu_sc as plsc

B = 16384
L = 50
EMBD = 64
NTOT = B * L
NW = 32
B_PER_W = NTOT // NW
CH = 320
NCHUNK = B_PER_W // CH
NBUF = 4

_mesh = plsc.VectorSubcoreMesh(core_axis_name="c", subcore_axis_name="s")


@functools.partial(
    pl.kernel,
    mesh=_mesh,
    out_type=jax.ShapeDtypeStruct((NTOT, EMBD), jnp.float32),
    scratch_types=[
        [pltpu.VMEM((CH,), jnp.int32) for _ in range(NBUF)],
        [pltpu.VMEM((CH, EMBD), jnp.float32) for _ in range(NBUF)],
        pltpu.SemaphoreType.DMA((NBUF,)),
        pltpu.SemaphoreType.DMA((NBUF,)),
    ],
    compiler_params=pltpu.CompilerParams(use_tc_tiling_on_sc=False),
)
def _gather_kernel(idx_hbm, table_hbm, out_hbm, idx_v, rows_v, gsem, ssem):
    wid = lax.axis_index("s") * 2 + lax.axis_index("c")
    base = wid * B_PER_W

    def start_gather(b, i):
        pltpu.sync_copy(idx_hbm.at[pl.ds(base + i * CH, CH)], idx_v[b])
        pltpu.async_copy(table_hbm.at[idx_v[b]], rows_v[b], gsem.at[b])

    def wait_gather(b, i):
        pltpu.make_async_copy(
            table_hbm.at[idx_v[b]], rows_v[b], gsem.at[b]).wait()

    def start_store(b, i):
        pltpu.async_copy(
            rows_v[b], out_hbm.at[pl.ds(base + i * CH, CH)], ssem.at[b])

    def wait_store(b, i):
        pltpu.make_async_copy(
            rows_v[b], out_hbm.at[pl.ds(base + i * CH, CH)],
            ssem.at[b]).wait()

    start_gather(0, 0)
    start_gather(1, 1)
    for k in (0, 1):
        wait_gather(k, k)
        start_store(k, k)
        start_gather(k + 2, k + 2)

    def group(j, carry):
        for k in range(NBUF):
            i = 2 + j * NBUF + k
            b = (2 + k) % NBUF
            bn = k
            wait_gather(b, i)
            start_store(b, i)
            wait_store(bn, i - 2)
            start_gather(bn, i + 2)
        return carry

    lax.fori_loop(0, (NCHUNK - 4) // NBUF, group, 0)

    for k in (0, 1):
        i = NCHUNK - 2 + k
        wait_gather((i % NBUF), i)
        start_store((i % NBUF), i)
    for k in range(NBUF):
        i = NCHUNK - 4 + k
        wait_store(i % NBUF, i)


def kernel(x, table):
    idx = x.reshape(NTOT)
    out = _gather_kernel(idx, table)
    return out.reshape(B, L, EMBD)
